# Initial kernel scaffold; baseline (speedup 1.0000x reference)
import jax
import jax.numpy as jnp
from jax.experimental import pallas as pl


def kernel(nodes, distance, edges_padding, W1, b1, W2, b2, a, b, Wself, Wg, bg, receivers, senders):
    raise NotImplementedError("write your pallas kernel here")


# speedup vs baseline: 2.5976x; 2.5976x over previous
import functools

import jax
import jax.numpy as jnp
from jax import lax
from jax.experimental import pallas as pl
from jax.experimental.pallas import tpu as pltpu, tpu_sc as plsc

_N = 10000
_E = 320000
_DF = 128
_NB = 64
_CH = 80


def _gather_rows(table, idx):
    info = plsc.get_sparse_core_info()
    nw = info.num_cores * info.num_subcores
    per_w = _E // nw
    n_iter = per_w // _CH
    mesh = plsc.VectorSubcoreMesh(core_axis_name="c", subcore_axis_name="s")

    @functools.partial(
        pl.kernel,
        mesh=mesh,
        out_type=jax.ShapeDtypeStruct((_E, _DF), jnp.float32),
        scratch_types=[
            pltpu.VMEM((_CH,), jnp.int32),
            pltpu.VMEM((_CH, _DF), jnp.float32),
            pltpu.SemaphoreType.DMA,
        ],
    )
    def k(table_hbm, idx_hbm, out_hbm, idx_v, rows_v, sem):
        wid = lax.axis_index("s") * info.num_cores + lax.axis_index("c")
        base0 = wid * per_w

        def body(i, carry):
            base = pl.multiple_of(base0 + i * _CH, 8)
            pltpu.sync_copy(idx_hbm.at[pl.ds(base, _CH)], idx_v)
            pltpu.async_copy(table_hbm.at[idx_v], rows_v, sem).wait()
            pltpu.sync_copy(rows_v, out_hbm.at[pl.ds(base, _CH)])
            return carry

        lax.fori_loop(0, n_iter, body, 0)

    return k(table, idx)


def _scatter_add(vals, idx, zeros):
    info = plsc.get_sparse_core_info()
    nw = info.num_cores * info.num_subcores
    per_w = _E // nw
    n_iter = per_w // _CH
    mesh = plsc.VectorSubcoreMesh(core_axis_name="c", subcore_axis_name="s")
    rows_per_drain = _N // 10

    @functools.partial(
        pl.kernel,
        mesh=mesh,
        out_type=(
            jax.ShapeDtypeStruct((_N, _DF), jnp.float32),
            jax.ShapeDtypeStruct((_N, _DF), jnp.float32),
        ),
        scratch_types=[
            pltpu.VMEM((_CH,), jnp.int32),
            pltpu.VMEM((_CH, _DF), jnp.float32),
            pltpu.VMEM_SHARED((_N, _DF), jnp.float32),
        ],
    )
    def k(vals_hbm, idx_hbm, zeros_hbm, out0_hbm, out1_hbm, idx_v, rows_v, acc):
        cid = lax.axis_index("c")
        sid = lax.axis_index("s")
        wid = sid * info.num_cores + cid
        base0 = wid * per_w

        @pl.when(sid == 0)
        def _():
            pltpu.sync_copy(zeros_hbm, acc)

        plsc.subcore_barrier()

        def body(i, carry):
            base = pl.multiple_of(base0 + i * _CH, 8)
            pltpu.sync_copy(idx_hbm.at[pl.ds(base, _CH)], idx_v)
            pltpu.sync_copy(vals_hbm.at[pl.ds(base, _CH)], rows_v)
            pltpu.sync_copy(rows_v, acc.at[idx_v], add=True)
            return carry

        lax.fori_loop(0, n_iter, body, 0)
        plsc.subcore_barrier()

        @pl.when(sid < 10)
        def _():
            r0 = pl.multiple_of(sid * rows_per_drain, 8)

            @pl.when(cid == 0)
            def _():
                pltpu.sync_copy(acc.at[pl.ds(r0, rows_per_drain)],
                                out0_hbm.at[pl.ds(r0, rows_per_drain)])

            @pl.when(cid == 1)
            def _():
                pltpu.sync_copy(acc.at[pl.ds(r0, rows_per_drain)],
                                out1_hbm.at[pl.ds(r0, rows_per_drain)])

    return k(vals, idx, zeros)


def _edge_body(d_ref, pad_ref, nr_ref, ns_ref, w1_ref, b1_ref, w2_ref,
               b2_ref, a_ref, b_ref, lo_ref, hi_ref, norms_ref, nums_ref):
    d = d_ref[...]
    lo = lo_ref[...]
    hi = hi_ref[...]
    ind = jnp.where((lo - d) * (d - hi) > 0.0, 1.0, 0.0)

    w1 = w1_ref[...]
    b1 = b1_ref[...]
    hidden = jnp.maximum(d * w1 + b1, 0.0)
    mlp = jnp.dot(hidden, w2_ref[...],
                  preferred_element_type=jnp.float32) + b2_ref[...]

    hmax = jnp.maximum(jnp.maximum(b1, 0.0), jnp.maximum(w1 + b1, 0.0))
    ub = b2_ref[...] + jnp.sum(jnp.maximum(w2_ref[...] * hmax.T, 0.0),
                               axis=0, keepdims=True)
    expm = jnp.exp(mlp - ub)

    ac = jnp.clip(a_ref[0, 0], 0.0, 1.0)
    bc = jnp.abs(b_ref[0, 0])
    diff = jnp.abs(ac * nr_ref[...] - (1.0 - ac) * ns_ref[...])
    pds = jnp.power(diff, bc)

    norms_ref[...] = jnp.concatenate([ind, expm], axis=1)
    pad = pad_ref[...]
    nums_ref[...] = jnp.concatenate(
        [ind * pds[:, :_NB], expm * pds[:, _NB:]], axis=1) * pad


def _node_body(nodes_ref, n0_ref, n1_ref, m0_ref, m1_ref, wself_ref,
               wg_ref, bg_ref, out_ref):
    norms = n0_ref[...] + n1_ref[...]
    nums = m0_ref[...] + m1_ref[...]
    s = norms[:, :_NB]
    z = norms[:, _NB:]
    g_ind = nums[:, :_NB] / (s + 1e-5)
    g_mlp = jnp.where(z > 0.0, nums[:, _NB:] / jnp.where(z > 0.0, z, 1.0), 0.0)
    gathered = jnp.concatenate([g_ind, g_mlp], axis=1)
    out = jnp.dot(nodes_ref[...], wself_ref[...],
                  preferred_element_type=jnp.float32)
    out += jnp.dot(gathered, wg_ref[...],
                   preferred_element_type=jnp.float32) + bg_ref[...]
    out_ref[...] = jnp.maximum(out, 0.0)


@jax.jit
def kernel(nodes, distance, edges_padding, W1, b1, W2, b2, a, b, Wself, Wg,
           bg, receivers, senders):
    n = nodes.shape[0]
    e = distance.shape[0]
    nr = _gather_rows(nodes, receivers)
    ns = _gather_rows(nodes, senders)

    bins = jnp.linspace(0.0, 1.0, _NB + 1, dtype=jnp.float32)
    lo = bins[:-1].reshape(1, _NB)
    hi = bins[1:].reshape(1, _NB)
    eb = 512
    row = lambda i: (i, 0)
    fixed = lambda i: (0, 0)
    norms, nums = pl.pallas_call(
        _edge_body,
        grid=(e // eb,),
        in_specs=[
            pl.BlockSpec((eb, 1), row),
            pl.BlockSpec((eb, 1), row),
            pl.BlockSpec((eb, _DF), row),
            pl.BlockSpec((eb, _DF), row),
            pl.BlockSpec((1, _NB), fixed),
            pl.BlockSpec((1, _NB), fixed),
            pl.BlockSpec((_NB, _NB), fixed),
            pl.BlockSpec((1, _NB), fixed),
            pl.BlockSpec((1, 1), fixed),
            pl.BlockSpec((1, 1), fixed),
            pl.BlockSpec((1, _NB), fixed),
            pl.BlockSpec((1, _NB), fixed),
        ],
        out_specs=[pl.BlockSpec((eb, _DF), row), pl.BlockSpec((eb, _DF), row)],
        out_shape=[
            jax.ShapeDtypeStruct((e, _DF), jnp.float32),
            jax.ShapeDtypeStruct((e, _DF), jnp.float32),
        ],
    )(distance.reshape(e, 1), edges_padding.reshape(e, 1), nr, ns,
      W1, b1.reshape(1, _NB), W2, b2.reshape(1, _NB),
      a.reshape(1, 1), b.reshape(1, 1), lo, hi)

    zeros = jnp.zeros((n, _DF), dtype=jnp.float32)
    n0, n1 = _scatter_add(norms, receivers, zeros)
    m0, m1 = _scatter_add(nums, receivers, zeros)

    nb = 1000
    out = pl.pallas_call(
        _node_body,
        grid=(n // nb,),
        in_specs=[
            pl.BlockSpec((nb, _DF), row),
            pl.BlockSpec((nb, _DF), row),
            pl.BlockSpec((nb, _DF), row),
            pl.BlockSpec((nb, _DF), row),
            pl.BlockSpec((nb, _DF), row),
            pl.BlockSpec((_DF, _DF), fixed),
            pl.BlockSpec((_DF, _DF), fixed),
            pl.BlockSpec((1, _DF), fixed),
        ],
        out_specs=pl.BlockSpec((nb, _DF), row),
        out_shape=jax.ShapeDtypeStruct((n, _DF), jnp.float32),
    )(nodes, n0, n1, m0, m1, Wself, Wg, bg.reshape(1, _DF))
    return out
